# Initial kernel scaffold; baseline (speedup 1.0000x reference)
import functools

import jax
import jax.numpy as jnp
from jax import lax
from jax.experimental import pallas as pl
from jax.experimental.pallas import tpu as pltpu
from jax.experimental.pallas import tpu_sc as plsc

NUM_CLASSES = 1000
NUM_REL = 128
N_BOXES = 4096
N_PAIRS = 131072

NC = 2
NS = 16
L = 16
NW = NC * NS
PAIRS_PER_W = N_PAIRS // NW
CHUNK = 128
N_CHUNKS = PAIRS_PER_W // CHUNK
KGRP = NUM_REL // L


def _sc_prior(labels, subbox, objbox, sub_dist, obj_dist):
    mesh = plsc.VectorSubcoreMesh(core_axis_name="c", subcore_axis_name="s")

    @functools.partial(
        pl.kernel,
        mesh=mesh,
        out_type=jax.ShapeDtypeStruct((N_PAIRS, NUM_REL), jnp.float32),
        scratch_types=[
            pltpu.VMEM((N_BOXES,), jnp.int32),
            pltpu.VMEM((PAIRS_PER_W,), jnp.int32),
            pltpu.VMEM((PAIRS_PER_W,), jnp.int32),
            pltpu.VMEM((CHUNK,), jnp.int32),
            pltpu.VMEM((CHUNK,), jnp.int32),
            pltpu.VMEM((CHUNK, NUM_REL), jnp.float32),
            pltpu.VMEM((CHUNK, NUM_REL), jnp.float32),
            pltpu.VMEM((CHUNK, NUM_REL), jnp.float32),
            pltpu.SemaphoreType.DMA,
        ],
    )
    def k(labels_hbm, subbox_hbm, objbox_hbm, subd_hbm, objd_hbm, out_hbm,
          labels_v, subbox_v, objbox_v, slab_v, olab_v, subr_v, objr_v,
          prod_v, sem):
        wid = lax.axis_index("s") * NC + lax.axis_index("c")
        base = wid * PAIRS_PER_W
        pltpu.sync_copy(labels_hbm, labels_v)
        pltpu.sync_copy(subbox_hbm.at[pl.ds(base, PAIRS_PER_W)], subbox_v)
        pltpu.sync_copy(objbox_hbm.at[pl.ds(base, PAIRS_PER_W)], objbox_v)

        def chunk_body(c, carry):
            cb = c * CHUNK

            def lab_body(j, carry2):
                off = cb + j * L
                sb = subbox_v[pl.ds(off, L)]
                ob = objbox_v[pl.ds(off, L)]
                slab_v[pl.ds(j * L, L)] = plsc.load_gather(labels_v, [sb])
                olab_v[pl.ds(j * L, L)] = plsc.load_gather(labels_v, [ob])
                return carry2

            lax.fori_loop(0, CHUNK // L, lab_body, 0)

            cp_s = pltpu.async_copy(subd_hbm.at[slab_v], subr_v, sem)
            cp_o = pltpu.async_copy(objd_hbm.at[olab_v], objr_v, sem)
            cp_s.wait()
            cp_o.wait()

            def row_body(r, carry2):
                ps = []
                acc = None
                for g in range(KGRP):
                    s = subr_v[r, pl.ds(g * L, L)]
                    o = objr_v[r, pl.ds(g * L, L)]
                    p = s * o
                    ps.append(p)
                    a = jnp.abs(p)
                    acc = a if acc is None else acc + a
                norm = jnp.sum(acc)
                normv = jnp.broadcast_to(norm, (L,))
                normv = jnp.maximum(normv, jnp.full((L,), 1e-12, jnp.float32))
                invv = jnp.full((L,), 1.0, jnp.float32) / normv
                for g in range(KGRP):
                    prod_v[r, pl.ds(g * L, L)] = ps[g] * invv
                return carry2

            lax.fori_loop(0, CHUNK, row_body, 0)
            pltpu.sync_copy(prod_v, out_hbm.at[pl.ds(base + cb, CHUNK)])
            return carry

        lax.fori_loop(0, N_CHUNKS, chunk_body, 0)

    return k(labels, subbox, objbox, sub_dist, obj_dist)


def kernel(labels, rel_pair_idx, sub_distribution, obj_distribution):
    subbox = rel_pair_idx[:, 0]
    objbox = rel_pair_idx[:, 1]
    return _sc_prior(labels, subbox, objbox, sub_distribution,
                     obj_distribution)


# speedup vs baseline: 17.9630x; 17.9630x over previous
import functools

import jax
import jax.numpy as jnp
from jax import lax
from jax.experimental import pallas as pl
from jax.experimental.pallas import tpu as pltpu
from jax.experimental.pallas import tpu_sc as plsc

NUM_CLASSES = 1000
NUM_REL = 128
N_BOXES = 4096
N_PAIRS = 131072

NC = 2
NS = 16
L = 16
NW = NC * NS
PAIRS_PER_W = N_PAIRS // NW
CHUNK = 128
N_CHUNKS = PAIRS_PER_W // CHUNK
KGRP = NUM_REL // L


def _sc_prior(labels, subbox, objbox, sub_dist, obj_dist):
    mesh = plsc.VectorSubcoreMesh(core_axis_name="c", subcore_axis_name="s")

    @functools.partial(
        pl.kernel,
        mesh=mesh,
        compiler_params=pltpu.CompilerParams(needs_layout_passes=False),
        out_type=jax.ShapeDtypeStruct((N_PAIRS, NUM_REL), jnp.float32),
        scratch_types=[
            pltpu.VMEM((N_BOXES,), jnp.int32),
            pltpu.VMEM((PAIRS_PER_W,), jnp.int32),
            pltpu.VMEM((PAIRS_PER_W,), jnp.int32),
            pltpu.VMEM((CHUNK,), jnp.int32),
            pltpu.VMEM((CHUNK,), jnp.int32),
            pltpu.VMEM((CHUNK, NUM_REL), jnp.float32),
            pltpu.VMEM((CHUNK, NUM_REL), jnp.float32),
            pltpu.VMEM((CHUNK, NUM_REL), jnp.float32),
            pltpu.SemaphoreType.DMA,
        ],
    )
    def k(labels_hbm, subbox_hbm, objbox_hbm, subd_hbm, objd_hbm, out_hbm,
          labels_v, subbox_v, objbox_v, slab_v, olab_v, subr_v, objr_v,
          prod_v, sem):
        wid = lax.axis_index("s") * NC + lax.axis_index("c")
        base = wid * PAIRS_PER_W
        pltpu.sync_copy(labels_hbm, labels_v)
        pltpu.sync_copy(subbox_hbm.at[pl.ds(base, PAIRS_PER_W)], subbox_v)
        pltpu.sync_copy(objbox_hbm.at[pl.ds(base, PAIRS_PER_W)], objbox_v)

        def chunk_body(c, carry):
            cb = c * CHUNK

            def lab_body(j, carry2):
                off = cb + j * L
                sb = subbox_v[pl.ds(off, L)]
                ob = objbox_v[pl.ds(off, L)]
                slab_v[pl.ds(j * L, L)] = plsc.load_gather(labels_v, [sb])
                olab_v[pl.ds(j * L, L)] = plsc.load_gather(labels_v, [ob])
                return carry2

            lax.fori_loop(0, CHUNK // L, lab_body, 0)

            cp_s = pltpu.async_copy(subd_hbm.at[slab_v], subr_v, sem)
            cp_o = pltpu.async_copy(objd_hbm.at[olab_v], objr_v, sem)
            cp_s.wait()
            cp_o.wait()

            def row_body(r, carry2):
                ps = []
                acc = None
                for g in range(KGRP):
                    s = subr_v[r, pl.ds(g * L, L)]
                    o = objr_v[r, pl.ds(g * L, L)]
                    p = s * o
                    ps.append(p)
                    a = jnp.abs(p)
                    acc = a if acc is None else acc + a
                norm = jnp.sum(acc)
                normv = jnp.broadcast_to(norm, (L,))
                normv = jnp.maximum(normv, jnp.full((L,), 1e-12, jnp.float32))
                invv = jnp.full((L,), 1.0, jnp.float32) / normv
                for g in range(KGRP):
                    prod_v[r, pl.ds(g * L, L)] = ps[g] * invv
                return carry2

            lax.fori_loop(0, CHUNK, row_body, 0)
            pltpu.sync_copy(prod_v, out_hbm.at[pl.ds(base + cb, CHUNK)])
            return carry

        lax.fori_loop(0, N_CHUNKS, chunk_body, 0)

    return k(labels, subbox, objbox, sub_dist, obj_dist)


def kernel(labels, rel_pair_idx, sub_distribution, obj_distribution):
    subbox = rel_pair_idx[:, 0]
    objbox = rel_pair_idx[:, 1]
    return _sc_prior(labels, subbox, objbox, sub_distribution,
                     obj_distribution)


# speedup vs baseline: 21.8400x; 1.2158x over previous
import functools

import jax
import jax.numpy as jnp
from jax import lax
from jax.experimental import pallas as pl
from jax.experimental.pallas import tpu as pltpu
from jax.experimental.pallas import tpu_sc as plsc

NUM_CLASSES = 1000
NUM_REL = 128
N_BOXES = 4096
N_PAIRS = 131072

NC = 2
NS = 16
L = 16
NW = NC * NS
PAIRS_PER_W = N_PAIRS // NW
CHUNK = 128
N_CHUNKS = PAIRS_PER_W // CHUNK
KGRP = NUM_REL // L


def _sc_prior(labels, subbox, objbox, sub_dist, obj_dist):
    mesh = plsc.VectorSubcoreMesh(core_axis_name="c", subcore_axis_name="s")

    @functools.partial(
        pl.kernel,
        mesh=mesh,
        compiler_params=pltpu.CompilerParams(needs_layout_passes=False),
        out_type=jax.ShapeDtypeStruct((N_PAIRS, NUM_REL), jnp.float32),
        scratch_types=[
            pltpu.VMEM((N_BOXES,), jnp.int32),
            pltpu.VMEM((PAIRS_PER_W,), jnp.int32),
            pltpu.VMEM((PAIRS_PER_W,), jnp.int32),
            pltpu.VMEM((CHUNK,), jnp.int32),
            pltpu.VMEM((CHUNK,), jnp.int32),
            pltpu.VMEM((CHUNK,), jnp.int32),
            pltpu.VMEM((CHUNK,), jnp.int32),
            pltpu.VMEM((CHUNK, NUM_REL), jnp.float32),
            pltpu.VMEM((CHUNK, NUM_REL), jnp.float32),
            pltpu.VMEM((CHUNK, NUM_REL), jnp.float32),
            pltpu.VMEM((CHUNK, NUM_REL), jnp.float32),
            pltpu.VMEM((CHUNK, NUM_REL), jnp.float32),
            pltpu.VMEM((CHUNK, NUM_REL), jnp.float32),
            pltpu.SemaphoreType.DMA,
            pltpu.SemaphoreType.DMA,
            pltpu.SemaphoreType.DMA,
            pltpu.SemaphoreType.DMA,
        ],
    )
    def k(labels_hbm, subbox_hbm, objbox_hbm, subd_hbm, objd_hbm, out_hbm,
          labels_v, subbox_v, objbox_v, slab0, olab0, slab1, olab1,
          subr0, objr0, subr1, objr1, prod0, prod1,
          sem_g0, sem_g1, sem_o0, sem_o1):
        wid = lax.axis_index("s") * NC + lax.axis_index("c")
        base = wid * PAIRS_PER_W
        pltpu.sync_copy(labels_hbm, labels_v)
        pltpu.sync_copy(subbox_hbm.at[pl.ds(base, PAIRS_PER_W)], subbox_v)
        pltpu.sync_copy(objbox_hbm.at[pl.ds(base, PAIRS_PER_W)], objbox_v)

        bufs = (
            (slab0, olab0, subr0, objr0, prod0, sem_g0, sem_o0),
            (slab1, olab1, subr1, objr1, prod1, sem_g1, sem_o1),
        )

        def labels_for(c, slab, olab):
            cb = c * CHUNK

            @plsc.parallel_loop(0, CHUNK // L)
            def lab_body(j):
                off = cb + j * L
                sb = subbox_v[pl.ds(off, L)]
                ob = objbox_v[pl.ds(off, L)]
                slab[pl.ds(j * L, L)] = plsc.load_gather(labels_v, [sb])
                olab[pl.ds(j * L, L)] = plsc.load_gather(labels_v, [ob])

        def start_gathers(slab, olab, subr, objr, sem):
            pltpu.async_copy(subd_hbm.at[slab], subr, sem)
            pltpu.async_copy(objd_hbm.at[olab], objr, sem)

        for b in range(2):
            slab, olab, subr, objr, _, sem_g, _ = bufs[b]
            labels_for(b, slab, olab)
            start_gathers(slab, olab, subr, objr, sem_g)

        def pair_body(j, carry):
            for b in range(2):
                c = 2 * j + b
                slab, olab, subr, objr, prod, sem_g, sem_o = bufs[b]
                pltpu.make_async_copy(subd_hbm.at[slab], subr, sem_g).wait()
                pltpu.make_async_copy(objd_hbm.at[olab], objr, sem_g).wait()

                @pl.when(j > 0)
                def _():
                    pltpu.make_async_copy(
                        prod, out_hbm.at[pl.ds(base, CHUNK)], sem_o).wait()

                @plsc.parallel_loop(0, CHUNK, unroll=4)
                def row_body(r):
                    ps = []
                    acc = None
                    for g in range(KGRP):
                        s = subr[r, pl.ds(g * L, L)]
                        o = objr[r, pl.ds(g * L, L)]
                        p = s * o
                        ps.append(p)
                        a = jnp.abs(p)
                        acc = a if acc is None else acc + a
                    norm = jnp.sum(acc)
                    normv = jnp.broadcast_to(norm, (L,))
                    normv = jnp.maximum(
                        normv, jnp.full((L,), 1e-12, jnp.float32))
                    invv = jnp.full((L,), 1.0, jnp.float32) / normv
                    for g in range(KGRP):
                        prod[r, pl.ds(g * L, L)] = ps[g] * invv

                pltpu.async_copy(
                    prod, out_hbm.at[pl.ds(base + c * CHUNK, CHUNK)], sem_o)

                @pl.when(c + 2 < N_CHUNKS)
                def _():
                    labels_for(c + 2, slab, olab)
                    start_gathers(slab, olab, subr, objr, sem_g)

            return carry

        lax.fori_loop(0, N_CHUNKS // 2, pair_body, 0)

        for b in range(2):
            _, _, _, _, prod, _, sem_o = bufs[b]
            pltpu.make_async_copy(
                prod, out_hbm.at[pl.ds(base, CHUNK)], sem_o).wait()

    return k(labels, subbox, objbox, sub_dist, obj_dist)


def kernel(labels, rel_pair_idx, sub_distribution, obj_distribution):
    subbox = rel_pair_idx[:, 0]
    objbox = rel_pair_idx[:, 1]
    return _sc_prior(labels, subbox, objbox, sub_distribution,
                     obj_distribution)


# speedup vs baseline: 32.9934x; 1.5107x over previous
import functools

import jax
import jax.numpy as jnp
from jax import lax
from jax.experimental import pallas as pl
from jax.experimental.pallas import tpu as pltpu
from jax.experimental.pallas import tpu_sc as plsc

NUM_CLASSES = 1000
NUM_REL = 128
N_BOXES = 4096
N_PAIRS = 131072

NC = 2
NS = 16
L = 16
NW = NC * NS
PAIRS_PER_W = N_PAIRS // NW
CHUNK = 128
N_CHUNKS = PAIRS_PER_W // CHUNK
KGRP = NUM_REL // L


def _sc_prior(labels, subbox, objbox, sub_dist, obj_dist):
    mesh = plsc.VectorSubcoreMesh(core_axis_name="c", subcore_axis_name="s")

    @functools.partial(
        pl.kernel,
        mesh=mesh,
        compiler_params=pltpu.CompilerParams(needs_layout_passes=False),
        out_type=jax.ShapeDtypeStruct((N_PAIRS, NUM_REL), jnp.float32),
        scratch_types=[
            pltpu.VMEM_SHARED((NUM_CLASSES, NUM_REL), jnp.float32),
            pltpu.VMEM_SHARED((NUM_CLASSES, NUM_REL), jnp.float32),
            pltpu.VMEM((N_BOXES,), jnp.int32),
            pltpu.VMEM((PAIRS_PER_W,), jnp.int32),
            pltpu.VMEM((PAIRS_PER_W,), jnp.int32),
            pltpu.VMEM((CHUNK,), jnp.int32),
            pltpu.VMEM((CHUNK,), jnp.int32),
            pltpu.VMEM((CHUNK,), jnp.int32),
            pltpu.VMEM((CHUNK,), jnp.int32),
            pltpu.VMEM((CHUNK, NUM_REL), jnp.float32),
            pltpu.VMEM((CHUNK, NUM_REL), jnp.float32),
            pltpu.VMEM((CHUNK, NUM_REL), jnp.float32),
            pltpu.VMEM((CHUNK, NUM_REL), jnp.float32),
            pltpu.VMEM((CHUNK, NUM_REL), jnp.float32),
            pltpu.VMEM((CHUNK, NUM_REL), jnp.float32),
            pltpu.SemaphoreType.DMA,
            pltpu.SemaphoreType.DMA,
            pltpu.SemaphoreType.DMA,
            pltpu.SemaphoreType.DMA,
        ],
    )
    def k(labels_hbm, subbox_hbm, objbox_hbm, subd_hbm, objd_hbm, out_hbm,
          subd_sh, objd_sh,
          labels_v, subbox_v, objbox_v, slab0, olab0, slab1, olab1,
          subr0, objr0, subr1, objr1, prod0, prod1,
          sem_g0, sem_g1, sem_o0, sem_o1):
        sid = lax.axis_index("s")
        wid = sid * NC + lax.axis_index("c")
        base = wid * PAIRS_PER_W

        @pl.when(sid == 0)
        def _():
            pltpu.sync_copy(subd_hbm, subd_sh)
            pltpu.sync_copy(objd_hbm, objd_sh)

        pltpu.sync_copy(labels_hbm, labels_v)
        pltpu.sync_copy(subbox_hbm.at[pl.ds(base, PAIRS_PER_W)], subbox_v)
        pltpu.sync_copy(objbox_hbm.at[pl.ds(base, PAIRS_PER_W)], objbox_v)
        plsc.subcore_barrier()

        bufs = (
            (slab0, olab0, subr0, objr0, prod0, sem_g0, sem_o0),
            (slab1, olab1, subr1, objr1, prod1, sem_g1, sem_o1),
        )

        def labels_for(c, slab, olab):
            cb = c * CHUNK

            @plsc.parallel_loop(0, CHUNK // L)
            def lab_body(j):
                off = cb + j * L
                sb = subbox_v[pl.ds(off, L)]
                ob = objbox_v[pl.ds(off, L)]
                slab[pl.ds(j * L, L)] = plsc.load_gather(labels_v, [sb])
                olab[pl.ds(j * L, L)] = plsc.load_gather(labels_v, [ob])

        def start_gathers(slab, olab, subr, objr, sem):
            pltpu.async_copy(subd_sh.at[slab], subr, sem)
            pltpu.async_copy(objd_sh.at[olab], objr, sem)

        for b in range(2):
            slab, olab, subr, objr, _, sem_g, _ = bufs[b]
            labels_for(b, slab, olab)
            start_gathers(slab, olab, subr, objr, sem_g)

        def pair_body(j, carry):
            for b in range(2):
                c = 2 * j + b
                slab, olab, subr, objr, prod, sem_g, sem_o = bufs[b]
                pltpu.make_async_copy(subd_sh.at[slab], subr, sem_g).wait()
                pltpu.make_async_copy(objd_sh.at[olab], objr, sem_g).wait()

                @pl.when(j > 0)
                def _():
                    pltpu.make_async_copy(
                        prod, out_hbm.at[pl.ds(base, CHUNK)], sem_o).wait()

                @plsc.parallel_loop(0, CHUNK, unroll=4)
                def row_body(r):
                    ps = []
                    acc = None
                    for g in range(KGRP):
                        s = subr[r, pl.ds(g * L, L)]
                        o = objr[r, pl.ds(g * L, L)]
                        p = s * o
                        ps.append(p)
                        a = jnp.abs(p)
                        acc = a if acc is None else acc + a
                    norm = jnp.sum(acc)
                    normv = jnp.broadcast_to(norm, (L,))
                    normv = jnp.maximum(
                        normv, jnp.full((L,), 1e-12, jnp.float32))
                    invv = jnp.full((L,), 1.0, jnp.float32) / normv
                    for g in range(KGRP):
                        prod[r, pl.ds(g * L, L)] = ps[g] * invv

                pltpu.async_copy(
                    prod, out_hbm.at[pl.ds(base + c * CHUNK, CHUNK)], sem_o)

                @pl.when(c + 2 < N_CHUNKS)
                def _():
                    labels_for(c + 2, slab, olab)
                    start_gathers(slab, olab, subr, objr, sem_g)

            return carry

        lax.fori_loop(0, N_CHUNKS // 2, pair_body, 0)

        for b in range(2):
            _, _, _, _, prod, _, sem_o = bufs[b]
            pltpu.make_async_copy(
                prod, out_hbm.at[pl.ds(base, CHUNK)], sem_o).wait()

    return k(labels, subbox, objbox, sub_dist, obj_dist)


def kernel(labels, rel_pair_idx, sub_distribution, obj_distribution):
    subbox = rel_pair_idx[:, 0]
    objbox = rel_pair_idx[:, 1]
    return _sc_prior(labels, subbox, objbox, sub_distribution,
                     obj_distribution)
